# baseline (speedup 1.0000x reference)
import jax
import jax.numpy as jnp
from jax import lax
from jax.experimental import pallas as pl
from jax.experimental.pallas import tpu as pltpu
from jax.experimental.pallas import tpu_sc as plsc

B, L = 16384, 50
N = B * L
ASN_D = 19
GEO_V, GEO_D = 252, 8
OUT_D = 32

NC, NS = 2, 16
NW = NC * NS
PER_W = N // NW
T = 1024
NCHUNK = PER_W // T
G = T // 16
IDX_ROWS = T // 128
NBUF = 2


def _body_fixed(asn_table, geo_table, ips_table, in3d, asn_idx, out,
                inb0, inb1, aidx0, aidx1, gbuf0, gbuf1,
                outb0, outb1, geo_tab_v, ips_tab_v,
                gsem0, gsem1, osem0, osem1):
    pltpu.sync_copy(geo_table, geo_tab_v)
    pltpu.sync_copy(ips_table, ips_tab_v)
    _body_inner(asn_table, in3d, asn_idx, out,
                [inb0, inb1], [aidx0, aidx1], [gbuf0, gbuf1],
                [outb0, outb1], geo_tab_v, ips_tab_v,
                [gsem0, gsem1], [osem0, osem1])


def _body_inner(asn_table, in3d, asn_idx, out, inb, aidx, gbuf, outb,
                geo_tab_v, ips_tab_v, gsem, osem):
    wid = lax.axis_index("s") * NC + lax.axis_index("c")
    wc0 = wid * NCHUNK
    iota = lax.iota(jnp.int32, 16)

    gather_descs = [None] * NCHUNK
    out_descs = [None] * NCHUNK

    def stage_and_fire(j):
        b = j % NBUF
        c = wc0 + j
        pltpu.sync_copy(in3d.at[c], inb[b])
        pltpu.sync_copy(asn_idx.at[pl.ds(pl.multiple_of(c * IDX_ROWS, 8),
                                         IDX_ROWS)], aidx[b])
        descs = []
        for r in range(IDX_ROWS):
            descs.append(pltpu.async_copy(
                asn_table.at[aidx[b].at[r]],
                gbuf[b].at[pl.ds(r * 128, 128)],
                gsem[b]))
        gather_descs[j] = descs

    def assemble(j):
        b = j % NBUF

        def group_body(g, carry):
            t0 = g * 16
            rows = iota + t0
            latv = plsc.bitcast(inb[b][2, pl.ds(t0, 16)], jnp.float32)
            lonv = plsc.bitcast(inb[b][3, pl.ds(t0, 16)], jnp.float32)
            plsc.store_scatter(outb[b], [rows, jnp.zeros((16,), jnp.int32)],
                               latv)
            plsc.store_scatter(outb[b], [rows, jnp.ones((16,), jnp.int32)],
                               lonv)
            for col in range(ASN_D):
                v = plsc.load_gather(gbuf[b],
                                     [rows, jnp.full((16,), col, jnp.int32)])
                plsc.store_scatter(outb[b],
                                   [rows, jnp.full((16,), 2 + col,
                                                   jnp.int32)], v)
            gi = inb[b][0, pl.ds(t0, 16)]
            for col in range(GEO_D):
                v = plsc.load_gather(geo_tab_v,
                                     [gi, jnp.full((16,), col, jnp.int32)])
                plsc.store_scatter(outb[b],
                                   [rows, jnp.full((16,), 21 + col,
                                                   jnp.int32)], v)
            pi = inb[b][1, pl.ds(t0, 16)]
            for col in range(3):
                v = plsc.load_gather(ips_tab_v,
                                     [pi, jnp.full((16,), col, jnp.int32)])
                plsc.store_scatter(outb[b],
                                   [rows, jnp.full((16,), 29 + col,
                                                   jnp.int32)], v)
            return carry

        lax.fori_loop(0, G, group_body, 0)

    stage_and_fire(0)
    for j in range(NCHUNK):
        b = j % NBUF
        if j + 1 < NCHUNK:
            if j + 1 >= NBUF:
                out_descs[j - 1].wait()
            stage_and_fire(j + 1)
        for d in gather_descs[j]:
            d.wait()
        assemble(j)
        base = (wc0 + j) * T
        out_descs[j] = pltpu.async_copy(
            outb[b], out.at[pl.ds(base, T)], osem[b])
    for j in range(NCHUNK - 2, NCHUNK):
        out_descs[j].wait()


@jax.jit
def _run(asn_table, geo_table, ips_table, in3d, asn_idx):
    mesh = plsc.VectorSubcoreMesh(core_axis_name="c", subcore_axis_name="s")
    return pl.kernel(
        _body_fixed,
        out_type=jax.ShapeDtypeStruct((N, OUT_D), jnp.float32),
        mesh=mesh,
        compiler_params=pltpu.CompilerParams(
            needs_layout_passes=False, use_tc_tiling_on_sc=False),
        scratch_types=(
            [pltpu.VMEM((4, T), jnp.int32) for _ in range(NBUF)]
            + [pltpu.VMEM((IDX_ROWS, 128), jnp.int32) for _ in range(NBUF)]
            + [pltpu.VMEM((T, 24), jnp.float32) for _ in range(NBUF)]
            + [pltpu.VMEM((T, OUT_D), jnp.float32) for _ in range(NBUF)]
            + [pltpu.VMEM((GEO_V, GEO_D), jnp.float32),
               pltpu.VMEM((8, 4), jnp.float32)]
            + [pltpu.SemaphoreType.DMA for _ in range(2 * NBUF)]
        ),
    )(asn_table, geo_table, ips_table, in3d, asn_idx)


def kernel(x_lat, x_long, x_asn, x_geo_cc, x_ip_source,
           asn_table, geo_cc_table, ip_source_table):
    asn_idx = x_asn.reshape(N // 128, 128).astype(jnp.int32)
    geo_i = x_geo_cc.reshape(N).astype(jnp.int32)
    ips_i = x_ip_source.reshape(N).astype(jnp.int32)
    lat_b = lax.bitcast_convert_type(x_lat.reshape(N), jnp.int32)
    lon_b = lax.bitcast_convert_type(x_long.reshape(N), jnp.int32)
    in3d = (jnp.stack([geo_i, ips_i, lat_b, lon_b], axis=0)
            .reshape(4, N // T, T).transpose(1, 0, 2))
    ips_pad = jnp.pad(ip_source_table, ((0, 3), (0, 1)))
    asn_pad = jnp.pad(asn_table, ((0, 0), (0, 24 - ASN_D)))
    out = _run(asn_pad, geo_cc_table, ips_pad, in3d, asn_idx)
    return out.reshape(B, L, OUT_D)


# speedup vs baseline: 1.2588x; 1.2588x over previous
import jax
import jax.numpy as jnp
from jax import lax
from jax.experimental import pallas as pl
from jax.experimental.pallas import tpu as pltpu
from jax.experimental.pallas import tpu_sc as plsc

B, L = 16384, 50
N = B * L
ASN_D = 19
GEO_V, GEO_D = 252, 8
OUT_D = 32

NC, NS = 2, 16
NW = NC * NS
PER_W = N // NW
T = 1024
NCHUNK = PER_W // T
G = T // 16
IDX_ROWS = T // 128
NBUF = 3


def _body_fixed(asn_table, geo_table, ips_table, in3d, asn_idx, out,
                inb0, inb1, inb2, aidx0, aidx1, aidx2,
                outa0, outa1, outa2, outc0, outc1, outc2,
                geo_tab_v, ips_tab_v,
                gsem0, gsem1, gsem2, osem0, osem1, osem2):
    pltpu.sync_copy(geo_table, geo_tab_v)
    pltpu.sync_copy(ips_table, ips_tab_v)
    _body_inner(asn_table, in3d, asn_idx, out,
                [inb0, inb1, inb2], [aidx0, aidx1, aidx2],
                [outa0, outa1, outa2], [outc0, outc1, outc2],
                geo_tab_v, ips_tab_v,
                [gsem0, gsem1, gsem2], [osem0, osem1, osem2])


def _body_inner(asn_table, in3d, asn_idx, out, inb, aidx, outa, outc,
                geo_tab_v, ips_tab_v, gsem, osem):
    wid = lax.axis_index("s") * NC + lax.axis_index("c")
    wc0 = wid * NCHUNK
    iota = lax.iota(jnp.int32, 16)

    gather_descs = [None] * NCHUNK
    out_descs = [None] * NCHUNK

    def stage_and_fire(j):
        b = j % NBUF
        c = wc0 + j
        pltpu.sync_copy(in3d.at[c], inb[b])
        pltpu.sync_copy(asn_idx.at[pl.ds(pl.multiple_of(c * IDX_ROWS, 8),
                                         IDX_ROWS)], aidx[b])
        descs = []
        for r in range(IDX_ROWS):
            descs.append(pltpu.async_copy(
                asn_table.at[aidx[b].at[r]],
                outa[b].at[pl.ds(r * 128, 128)],
                gsem[b]))
        gather_descs[j] = descs

    def assemble(j):
        b = j % NBUF

        def group_body(g, carry):
            t0 = g * 16
            rows = iota + t0
            latv = plsc.bitcast(inb[b][2, pl.ds(t0, 16)], jnp.float32)
            lonv = plsc.bitcast(inb[b][3, pl.ds(t0, 16)], jnp.float32)
            plsc.store_scatter(outa[b], [rows, jnp.zeros((16,), jnp.int32)],
                               latv)
            plsc.store_scatter(outa[b], [rows, jnp.ones((16,), jnp.int32)],
                               lonv)
            gi = inb[b][0, pl.ds(t0, 16)]
            for col in range(GEO_D):
                dst = outa[b] if col < 3 else outc[b]
                dcol = 21 + col if col < 3 else col - 3
                v = plsc.load_gather(geo_tab_v,
                                     [gi, jnp.full((16,), col, jnp.int32)])
                plsc.store_scatter(dst,
                                   [rows, jnp.full((16,), dcol,
                                                   jnp.int32)], v)
            pi = inb[b][1, pl.ds(t0, 16)]
            for col in range(3):
                v = plsc.load_gather(ips_tab_v,
                                     [pi, jnp.full((16,), col, jnp.int32)])
                plsc.store_scatter(outc[b],
                                   [rows, jnp.full((16,), 5 + col,
                                                   jnp.int32)], v)
            return carry

        lax.fori_loop(0, G, group_body, 0)

    stage_and_fire(0)
    stage_and_fire(1)
    for j in range(NCHUNK):
        b = j % NBUF
        if j + 2 < NCHUNK:
            if j + 2 >= NBUF:
                for d in out_descs[j - 1]:
                    d.wait()
            stage_and_fire(j + 2)
        for d in gather_descs[j]:
            d.wait()
        assemble(j)
        base = (wc0 + j) * T
        out_descs[j] = (
            pltpu.async_copy(
                outa[b], out.at[pl.ds(base, T), pl.ds(0, 24)], osem[b]),
            pltpu.async_copy(
                outc[b], out.at[pl.ds(base, T), pl.ds(24, 8)], osem[b]))
    for j in range(NCHUNK - 2, NCHUNK):
        for d in out_descs[j]:
            d.wait()


@jax.jit
def _run(asn_table, geo_table, ips_table, in3d, asn_idx):
    mesh = plsc.VectorSubcoreMesh(core_axis_name="c", subcore_axis_name="s")
    return pl.kernel(
        _body_fixed,
        out_type=jax.ShapeDtypeStruct((N, OUT_D), jnp.float32),
        mesh=mesh,
        compiler_params=pltpu.CompilerParams(
            needs_layout_passes=False, use_tc_tiling_on_sc=False),
        scratch_types=(
            [pltpu.VMEM((4, T), jnp.int32) for _ in range(NBUF)]
            + [pltpu.VMEM((IDX_ROWS, 128), jnp.int32) for _ in range(NBUF)]
            + [pltpu.VMEM((T, 24), jnp.float32) for _ in range(NBUF)]
            + [pltpu.VMEM((T, 8), jnp.float32) for _ in range(NBUF)]
            + [pltpu.VMEM((GEO_V, GEO_D), jnp.float32),
               pltpu.VMEM((8, 4), jnp.float32)]
            + [pltpu.SemaphoreType.DMA for _ in range(2 * NBUF)]
        ),
    )(asn_table, geo_table, ips_table, in3d, asn_idx)


def kernel(x_lat, x_long, x_asn, x_geo_cc, x_ip_source,
           asn_table, geo_cc_table, ip_source_table):
    asn_idx = x_asn.reshape(N // 128, 128).astype(jnp.int32)
    geo_i = x_geo_cc.reshape(N).astype(jnp.int32)
    ips_i = x_ip_source.reshape(N).astype(jnp.int32)
    lat_b = lax.bitcast_convert_type(x_lat.reshape(N), jnp.int32)
    lon_b = lax.bitcast_convert_type(x_long.reshape(N), jnp.int32)
    in3d = (jnp.stack([geo_i, ips_i, lat_b, lon_b], axis=0)
            .reshape(4, N // T, T).transpose(1, 0, 2))
    ips_pad = jnp.pad(ip_source_table, ((0, 3), (0, 1)))
    asn_pad = jnp.pad(asn_table, ((0, 0), (2, 3)))
    out = _run(asn_pad, geo_cc_table, ips_pad, in3d, asn_idx)
    return out.reshape(B, L, OUT_D)


# speedup vs baseline: 1.2922x; 1.0265x over previous
import jax
import jax.numpy as jnp
from jax import lax
from jax.experimental import pallas as pl
from jax.experimental.pallas import tpu as pltpu
from jax.experimental.pallas import tpu_sc as plsc

B, L = 16384, 50
N = B * L
ASN_D = 19
GEO_V, GEO_D = 252, 8
OUT_D = 32

NC, NS = 2, 16
NW = NC * NS
PER_W = N // NW
T = 1024
NCHUNK = PER_W // T
G = T // 16
IDX_ROWS = T // 128
NBUF = 3


def _body_fixed(asn_table, geo_table, ips_table, lat, lon, geo_i, ips_i,
                asn_idx, out,
                latb0, latb1, latb2, lonb0, lonb1, lonb2,
                gib0, gib1, gib2, pib0, pib1, pib2,
                aidx0, aidx1, aidx2,
                outa0, outa1, outa2, outc0, outc1, outc2,
                geo_tab_v, ips_tab_v,
                gsem0, gsem1, gsem2, osem0, osem1, osem2):
    pltpu.sync_copy(geo_table, geo_tab_v)
    pltpu.sync_copy(ips_table, ips_tab_v)
    _body_inner(asn_table, lat, lon, geo_i, ips_i, asn_idx, out,
                [latb0, latb1, latb2], [lonb0, lonb1, lonb2],
                [gib0, gib1, gib2], [pib0, pib1, pib2],
                [aidx0, aidx1, aidx2],
                [outa0, outa1, outa2], [outc0, outc1, outc2],
                geo_tab_v, ips_tab_v,
                [gsem0, gsem1, gsem2], [osem0, osem1, osem2])


def _body_inner(asn_table, lat, lon, geo_i, ips_i, asn_idx, out,
                latb, lonb, gib, pib, aidx, outa, outc,
                geo_tab_v, ips_tab_v, gsem, osem):
    wid = lax.axis_index("s") * NC + lax.axis_index("c")
    wc0 = wid * NCHUNK
    iota = lax.iota(jnp.int32, 16)

    gather_descs = [None] * NCHUNK
    out_descs = [None] * NCHUNK

    def stage_and_fire(j):
        b = j % NBUF
        c = wc0 + j
        base = pl.multiple_of(c * T, 8)
        pltpu.sync_copy(lat.at[pl.ds(base, T)], latb[b])
        pltpu.sync_copy(lon.at[pl.ds(base, T)], lonb[b])
        pltpu.sync_copy(geo_i.at[pl.ds(base, T)], gib[b])
        pltpu.sync_copy(ips_i.at[pl.ds(base, T)], pib[b])
        pltpu.sync_copy(asn_idx.at[pl.ds(pl.multiple_of(c * IDX_ROWS, 8),
                                         IDX_ROWS)], aidx[b])
        descs = []
        for r in range(IDX_ROWS):
            descs.append(pltpu.async_copy(
                asn_table.at[aidx[b].at[r]],
                outa[b].at[pl.ds(r * 128, 128)],
                gsem[b]))
        gather_descs[j] = descs

    def assemble(j):
        b = j % NBUF

        def group_body(g, carry):
            t0 = g * 16
            rows = iota + t0
            latv = latb[b][pl.ds(t0, 16)]
            lonv = lonb[b][pl.ds(t0, 16)]
            plsc.store_scatter(outa[b], [rows, jnp.zeros((16,), jnp.int32)],
                               latv)
            plsc.store_scatter(outa[b], [rows, jnp.ones((16,), jnp.int32)],
                               lonv)
            gi = gib[b][pl.ds(t0, 16)]
            for col in range(GEO_D):
                dst = outa[b] if col < 3 else outc[b]
                dcol = 21 + col if col < 3 else col - 3
                v = plsc.load_gather(geo_tab_v,
                                     [gi, jnp.full((16,), col, jnp.int32)])
                plsc.store_scatter(dst,
                                   [rows, jnp.full((16,), dcol,
                                                   jnp.int32)], v)
            pi = pib[b][pl.ds(t0, 16)]
            for col in range(3):
                v = plsc.load_gather(ips_tab_v,
                                     [pi, jnp.full((16,), col, jnp.int32)])
                plsc.store_scatter(outc[b],
                                   [rows, jnp.full((16,), 5 + col,
                                                   jnp.int32)], v)
            return carry

        lax.fori_loop(0, G, group_body, 0)

    stage_and_fire(0)
    stage_and_fire(1)
    for j in range(NCHUNK):
        b = j % NBUF
        if j + 2 < NCHUNK:
            if j + 2 >= NBUF:
                for d in out_descs[j - 1]:
                    d.wait()
            stage_and_fire(j + 2)
        for d in gather_descs[j]:
            d.wait()
        assemble(j)
        base = (wc0 + j) * T
        out_descs[j] = (
            pltpu.async_copy(
                outa[b], out.at[pl.ds(base, T), pl.ds(0, 24)], osem[b]),
            pltpu.async_copy(
                outc[b], out.at[pl.ds(base, T), pl.ds(24, 8)], osem[b]))
    for j in range(NCHUNK - 2, NCHUNK):
        for d in out_descs[j]:
            d.wait()


@jax.jit
def _run(asn_table, geo_table, ips_table, lat, lon, geo_i, ips_i, asn_idx):
    mesh = plsc.VectorSubcoreMesh(core_axis_name="c", subcore_axis_name="s")
    return pl.kernel(
        _body_fixed,
        out_type=jax.ShapeDtypeStruct((N, OUT_D), jnp.float32),
        mesh=mesh,
        compiler_params=pltpu.CompilerParams(
            needs_layout_passes=False, use_tc_tiling_on_sc=False),
        scratch_types=(
            [pltpu.VMEM((T,), jnp.float32) for _ in range(NBUF)]
            + [pltpu.VMEM((T,), jnp.float32) for _ in range(NBUF)]
            + [pltpu.VMEM((T,), jnp.int32) for _ in range(NBUF)]
            + [pltpu.VMEM((T,), jnp.int32) for _ in range(NBUF)]
            + [pltpu.VMEM((IDX_ROWS, 128), jnp.int32) for _ in range(NBUF)]
            + [pltpu.VMEM((T, 24), jnp.float32) for _ in range(NBUF)]
            + [pltpu.VMEM((T, 8), jnp.float32) for _ in range(NBUF)]
            + [pltpu.VMEM((GEO_V, GEO_D), jnp.float32),
               pltpu.VMEM((8, 4), jnp.float32)]
            + [pltpu.SemaphoreType.DMA for _ in range(2 * NBUF)]
        ),
    )(asn_table, geo_table, ips_table, lat, lon, geo_i, ips_i, asn_idx)


def kernel(x_lat, x_long, x_asn, x_geo_cc, x_ip_source,
           asn_table, geo_cc_table, ip_source_table):
    asn_idx = x_asn.reshape(N // 128, 128).astype(jnp.int32)
    geo_i = x_geo_cc.reshape(N)
    ips_i = x_ip_source.reshape(N)
    lat = x_lat.reshape(N)
    lon = x_long.reshape(N)
    ips_pad = jnp.pad(ip_source_table, ((0, 3), (0, 1)))
    asn_pad = jnp.pad(asn_table, ((0, 0), (2, 3)))
    out = _run(asn_pad, geo_cc_table, ips_pad, lat, lon, geo_i, ips_i,
               asn_idx)
    return out.reshape(B, L, OUT_D)


# speedup vs baseline: 2.3936x; 1.8524x over previous
import jax
import jax.numpy as jnp
from jax import lax
from jax.experimental import pallas as pl
from jax.experimental.pallas import tpu as pltpu
from jax.experimental.pallas import tpu_sc as plsc

B, L = 16384, 50
N = B * L
ASN_D = 19
GEO_V, GEO_D = 252, 8
OUT_D = 32

NC, NS = 2, 16
NW = NC * NS
T = 512
CPL = B // T
NCHUNKS = N // T
PER_W = NCHUNKS // NW
G = T // 16
IDX_ROWS = T // 128
NBUF = 3


def _body_fixed(asn_table, geo_table, ips_table, lat, lon, geo_i, ips_i,
                asn_idx, out,
                latb0, latb1, latb2, lonb0, lonb1, lonb2,
                gib0, gib1, gib2, pib0, pib1, pib2,
                aidx0, aidx1, aidx2, gbuf0, gbuf1, gbuf2,
                sct0, sct1, sct2,
                geo_tab_v, ips_tab_v,
                gsem0, gsem1, gsem2, osem0, osem1, osem2):
    pltpu.sync_copy(geo_table, geo_tab_v)
    pltpu.sync_copy(ips_table, ips_tab_v)
    _body_inner(asn_table, lat, lon, geo_i, ips_i, asn_idx, out,
                [latb0, latb1, latb2], [lonb0, lonb1, lonb2],
                [gib0, gib1, gib2], [pib0, pib1, pib2],
                [aidx0, aidx1, aidx2], [gbuf0, gbuf1, gbuf2],
                [sct0, sct1, sct2],
                geo_tab_v, ips_tab_v,
                [gsem0, gsem1, gsem2], [osem0, osem1, osem2])


def _body_inner(asn_table, lat, lon, geo_i, ips_i, asn_idx, out,
                latb, lonb, gib, pib, aidx, gbuf, sct,
                geo_tab_v, ips_tab_v, gsem, osem):
    wid = lax.axis_index("s") * NC + lax.axis_index("c")
    kc0 = wid * PER_W
    iota = lax.iota(jnp.int32, 16)

    gather_descs = [None] * PER_W
    out_descs = [None] * PER_W

    def stage_and_fire(j):
        b = j % NBUF
        k = kc0 + j
        l = k // CPL
        b0 = pl.multiple_of((k % CPL) * T, 8)
        pltpu.sync_copy(lat.at[l, pl.ds(b0, T)], latb[b])
        pltpu.sync_copy(lon.at[l, pl.ds(b0, T)], lonb[b])
        pltpu.sync_copy(geo_i.at[l, pl.ds(b0, T)], gib[b])
        pltpu.sync_copy(ips_i.at[l, pl.ds(b0, T)], pib[b])
        pltpu.sync_copy(asn_idx.at[k], aidx[b])
        descs = []
        for r in range(IDX_ROWS):
            descs.append(pltpu.async_copy(
                asn_table.at[aidx[b].at[r]],
                gbuf[b].at[pl.ds(r * 128, 128)],
                gsem[b]))
        gather_descs[j] = descs

    def assemble(j):
        b = j % NBUF

        def group_body(g, carry):
            t0 = g * 16
            cols = iota + t0
            sct[b][0, pl.ds(t0, 16)] = latb[b][pl.ds(t0, 16)]
            sct[b][1, pl.ds(t0, 16)] = lonb[b][pl.ds(t0, 16)]
            for c in range(ASN_D):
                v = plsc.load_gather(gbuf[b],
                                     [cols, jnp.full((16,), 2 + c,
                                                     jnp.int32)])
                sct[b][2 + c, pl.ds(t0, 16)] = v
            gi = gib[b][pl.ds(t0, 16)]
            for c in range(GEO_D):
                v = plsc.load_gather(geo_tab_v,
                                     [gi, jnp.full((16,), c, jnp.int32)])
                sct[b][21 + c, pl.ds(t0, 16)] = v
            pi = pib[b][pl.ds(t0, 16)]
            for c in range(3):
                v = plsc.load_gather(ips_tab_v,
                                     [pi, jnp.full((16,), c, jnp.int32)])
                sct[b][29 + c, pl.ds(t0, 16)] = v
            return carry

        lax.fori_loop(0, G, group_body, 0)

    stage_and_fire(0)
    stage_and_fire(1)
    for j in range(PER_W):
        b = j % NBUF
        if j + 2 < PER_W:
            if j + 2 >= NBUF:
                out_descs[j - 1].wait()
            stage_and_fire(j + 2)
        for d in gather_descs[j]:
            d.wait()
        assemble(j)
        k = kc0 + j
        l = k // CPL
        b0 = pl.multiple_of((k % CPL) * T, 8)
        out_descs[j] = pltpu.async_copy(
            sct[b], out.at[l, :, pl.ds(b0, T)], osem[b])
    for j in range(PER_W - 2, PER_W):
        out_descs[j].wait()


@jax.jit
def _run(asn_table, geo_table, ips_table, lat, lon, geo_i, ips_i, asn_idx):
    mesh = plsc.VectorSubcoreMesh(core_axis_name="c", subcore_axis_name="s")
    return pl.kernel(
        _body_fixed,
        out_type=jax.ShapeDtypeStruct((L, OUT_D, B), jnp.float32),
        mesh=mesh,
        compiler_params=pltpu.CompilerParams(
            needs_layout_passes=False, use_tc_tiling_on_sc=False),
        scratch_types=(
            [pltpu.VMEM((T,), jnp.float32) for _ in range(NBUF)]
            + [pltpu.VMEM((T,), jnp.float32) for _ in range(NBUF)]
            + [pltpu.VMEM((T,), jnp.int32) for _ in range(NBUF)]
            + [pltpu.VMEM((T,), jnp.int32) for _ in range(NBUF)]
            + [pltpu.VMEM((IDX_ROWS, 128), jnp.int32) for _ in range(NBUF)]
            + [pltpu.VMEM((T, 24), jnp.float32) for _ in range(NBUF)]
            + [pltpu.VMEM((OUT_D, T), jnp.float32) for _ in range(NBUF)]
            + [pltpu.VMEM((GEO_V, GEO_D), jnp.float32),
               pltpu.VMEM((8, 4), jnp.float32)]
            + [pltpu.SemaphoreType.DMA for _ in range(2 * NBUF)]
        ),
    )(asn_table, geo_table, ips_table, lat, lon, geo_i, ips_i, asn_idx)


def kernel(x_lat, x_long, x_asn, x_geo_cc, x_ip_source,
           asn_table, geo_cc_table, ip_source_table):
    lat = x_lat.reshape(B, L).T
    lon = x_long.reshape(B, L).T
    geo_i = x_geo_cc.T
    ips_i = x_ip_source.T
    asn_idx = x_asn.astype(jnp.int32).T.reshape(NCHUNKS, IDX_ROWS, 128)
    ips_pad = jnp.pad(ip_source_table, ((0, 3), (0, 1)))
    asn_pad = jnp.pad(asn_table, ((0, 0), (2, 3)))
    out_t = _run(asn_pad, geo_cc_table, ips_pad, lat, lon, geo_i, ips_i,
                 asn_idx)
    return jnp.transpose(out_t, (2, 0, 1))


# speedup vs baseline: 2.7876x; 1.1646x over previous
import jax
import jax.numpy as jnp
from jax import lax
from jax.experimental import pallas as pl
from jax.experimental.pallas import tpu as pltpu
from jax.experimental.pallas import tpu_sc as plsc

B, L = 16384, 50
N = B * L
ASN_D = 19
GEO_V, GEO_D = 252, 8
OUT_D = 32

NC, NS = 2, 16
NW = NC * NS
T = 512
CPL = B // T
NCHUNKS = N // T
PER_W = NCHUNKS // NW
G = T // 16
IDX_ROWS = T // 128
NBUF = 3


def _body_fixed(asn_table, geo_table, ips_table, lat, lon, geo_i, ips_i,
                asn_idx, out,
                latb0, latb1, latb2, lonb0, lonb1, lonb2,
                gib0, gib1, gib2, pib0, pib1, pib2,
                aidx_all, gbuf0, gbuf1, gbuf2,
                sct0, sct1, sct2,
                geo_tab_v, ips_tab_v,
                gsem0, gsem1, gsem2, osem0, osem1, osem2,
                isem0, isem1, isem2):
    pltpu.sync_copy(geo_table, geo_tab_v)
    pltpu.sync_copy(ips_table, ips_tab_v)
    wid = lax.axis_index("s") * NC + lax.axis_index("c")
    kc0 = wid * PER_W
    pltpu.sync_copy(asn_idx.at[pl.ds(pl.multiple_of(kc0, 2), PER_W)],
                    aidx_all)
    _body_inner(asn_table, lat, lon, geo_i, ips_i, kc0, out,
                [latb0, latb1, latb2], [lonb0, lonb1, lonb2],
                [gib0, gib1, gib2], [pib0, pib1, pib2],
                aidx_all, [gbuf0, gbuf1, gbuf2],
                [sct0, sct1, sct2],
                geo_tab_v, ips_tab_v,
                [gsem0, gsem1, gsem2], [osem0, osem1, osem2],
                [isem0, isem1, isem2])


def _body_inner(asn_table, lat, lon, geo_i, ips_i, kc0, out,
                latb, lonb, gib, pib, aidx_all, gbuf, sct,
                geo_tab_v, ips_tab_v, gsem, osem, isem):
    iota = lax.iota(jnp.int32, 16)

    gather_descs = [None] * PER_W
    in_descs = [None] * PER_W
    out_descs = [None] * PER_W

    def stage_and_fire(j):
        b = j % NBUF
        k = kc0 + j
        l = k // CPL
        b0 = pl.multiple_of((k % CPL) * T, 8)
        in_descs[j] = [
            pltpu.async_copy(lat.at[l, pl.ds(b0, T)], latb[b], isem[b]),
            pltpu.async_copy(lon.at[l, pl.ds(b0, T)], lonb[b], isem[b]),
            pltpu.async_copy(geo_i.at[l, pl.ds(b0, T)], gib[b], isem[b]),
            pltpu.async_copy(ips_i.at[l, pl.ds(b0, T)], pib[b], isem[b]),
        ]
        descs = []
        for r in range(IDX_ROWS):
            descs.append(pltpu.async_copy(
                asn_table.at[aidx_all.at[j, r]],
                gbuf[b].at[pl.ds(r * 128, 128)],
                gsem[b]))
        gather_descs[j] = descs

    def assemble(j):
        b = j % NBUF

        def group_body(g, carry):
            t0 = g * 16
            cols = iota + t0
            sct[b][0, pl.ds(t0, 16)] = latb[b][pl.ds(t0, 16)]
            sct[b][1, pl.ds(t0, 16)] = lonb[b][pl.ds(t0, 16)]
            for c in range(ASN_D):
                v = plsc.load_gather(gbuf[b],
                                     [cols, jnp.full((16,), 2 + c,
                                                     jnp.int32)])
                sct[b][2 + c, pl.ds(t0, 16)] = v
            gi = gib[b][pl.ds(t0, 16)]
            for c in range(GEO_D):
                v = plsc.load_gather(geo_tab_v,
                                     [gi, jnp.full((16,), c, jnp.int32)])
                sct[b][21 + c, pl.ds(t0, 16)] = v
            pi = pib[b][pl.ds(t0, 16)]
            for c in range(3):
                v = plsc.load_gather(ips_tab_v,
                                     [pi, jnp.full((16,), c, jnp.int32)])
                sct[b][29 + c, pl.ds(t0, 16)] = v
            return carry

        lax.fori_loop(0, G, group_body, 0)

    stage_and_fire(0)
    stage_and_fire(1)
    for j in range(PER_W):
        b = j % NBUF
        if j + 2 < PER_W:
            if j + 2 >= NBUF:
                out_descs[j - 1].wait()
            stage_and_fire(j + 2)
        for d in gather_descs[j]:
            d.wait()
        for d in in_descs[j]:
            d.wait()
        assemble(j)
        k = kc0 + j
        l = k // CPL
        b0 = pl.multiple_of((k % CPL) * T, 8)
        out_descs[j] = pltpu.async_copy(
            sct[b], out.at[l, :, pl.ds(b0, T)], osem[b])
    for j in range(PER_W - 2, PER_W):
        out_descs[j].wait()


@jax.jit
def _run(asn_table, geo_table, ips_table, lat, lon, geo_i, ips_i, asn_idx):
    mesh = plsc.VectorSubcoreMesh(core_axis_name="c", subcore_axis_name="s")
    return pl.kernel(
        _body_fixed,
        out_type=jax.ShapeDtypeStruct((L, OUT_D, B), jnp.float32),
        mesh=mesh,
        compiler_params=pltpu.CompilerParams(
            needs_layout_passes=False, use_tc_tiling_on_sc=False),
        scratch_types=(
            [pltpu.VMEM((T,), jnp.float32) for _ in range(NBUF)]
            + [pltpu.VMEM((T,), jnp.float32) for _ in range(NBUF)]
            + [pltpu.VMEM((T,), jnp.int32) for _ in range(NBUF)]
            + [pltpu.VMEM((T,), jnp.int32) for _ in range(NBUF)]
            + [pltpu.VMEM((PER_W, IDX_ROWS, 128), jnp.int32)]
            + [pltpu.VMEM((T, 24), jnp.float32) for _ in range(NBUF)]
            + [pltpu.VMEM((OUT_D, T), jnp.float32) for _ in range(NBUF)]
            + [pltpu.VMEM((GEO_V, GEO_D), jnp.float32),
               pltpu.VMEM((8, 4), jnp.float32)]
            + [pltpu.SemaphoreType.DMA for _ in range(3 * NBUF)]
        ),
    )(asn_table, geo_table, ips_table, lat, lon, geo_i, ips_i, asn_idx)


def kernel(x_lat, x_long, x_asn, x_geo_cc, x_ip_source,
           asn_table, geo_cc_table, ip_source_table):
    lat = x_lat.reshape(B, L).T
    lon = x_long.reshape(B, L).T
    geo_i = x_geo_cc.T
    ips_i = x_ip_source.T
    asn_idx = x_asn.astype(jnp.int32).T.reshape(NCHUNKS, IDX_ROWS, 128)
    ips_pad = jnp.pad(ip_source_table, ((0, 3), (0, 1)))
    asn_pad = jnp.pad(asn_table, ((0, 0), (2, 3)))
    out_t = _run(asn_pad, geo_cc_table, ips_pad, lat, lon, geo_i, ips_i,
                 asn_idx)
    return jnp.transpose(out_t, (2, 0, 1))


# speedup vs baseline: 2.7894x; 1.0006x over previous
import jax
import jax.numpy as jnp
from jax import lax
from jax.experimental import layout as jex_layout
from jax.experimental import pallas as pl
from jax.experimental.pallas import tpu as pltpu
from jax.experimental.pallas import tpu_sc as plsc

B, L = 16384, 50
N = B * L
ASN_D = 19
GEO_V, GEO_D = 252, 8
OUT_D = 32

NC, NS = 2, 16
NW = NC * NS
T = 512
CPL = B // T
NCHUNKS = N // T
PER_W = NCHUNKS // NW
G = T // 16
IDX_ROWS = T // 128
NBUF = 3


def _body_fixed(asn_table, geo_table, ips_table, lat, lon, geo_i, ips_i,
                asn_idx, out,
                latb0, latb1, latb2, lonb0, lonb1, lonb2,
                gib0, gib1, gib2, pib0, pib1, pib2,
                aidx_all, gbuf0, gbuf1, gbuf2,
                sct0, sct1, sct2,
                geo_tab_v, ips_tab_v,
                gsem0, gsem1, gsem2, osem0, osem1, osem2,
                isem0, isem1, isem2):
    pltpu.sync_copy(geo_table, geo_tab_v)
    pltpu.sync_copy(ips_table, ips_tab_v)
    wid = lax.axis_index("s") * NC + lax.axis_index("c")
    kc0 = wid * PER_W
    pltpu.sync_copy(asn_idx.at[pl.ds(pl.multiple_of(kc0, 2), PER_W)],
                    aidx_all)
    _body_inner(asn_table, lat, lon, geo_i, ips_i, kc0, out,
                [latb0, latb1, latb2], [lonb0, lonb1, lonb2],
                [gib0, gib1, gib2], [pib0, pib1, pib2],
                aidx_all, [gbuf0, gbuf1, gbuf2],
                [sct0, sct1, sct2],
                geo_tab_v, ips_tab_v,
                [gsem0, gsem1, gsem2], [osem0, osem1, osem2],
                [isem0, isem1, isem2])


def _body_inner(asn_table, lat, lon, geo_i, ips_i, kc0, out,
                latb, lonb, gib, pib, aidx_all, gbuf, sct,
                geo_tab_v, ips_tab_v, gsem, osem, isem):
    iota = lax.iota(jnp.int32, 16)

    gather_descs = [None] * PER_W
    in_descs = [None] * PER_W
    out_descs = [None] * PER_W

    def stage_and_fire(j):
        b = j % NBUF
        k = kc0 + j
        l = k // CPL
        b0 = pl.multiple_of((k % CPL) * T, 8)
        in_descs[j] = [
            pltpu.async_copy(lat.at[l, pl.ds(b0, T)], latb[b], isem[b]),
            pltpu.async_copy(lon.at[l, pl.ds(b0, T)], lonb[b], isem[b]),
            pltpu.async_copy(geo_i.at[l, pl.ds(b0, T)], gib[b], isem[b]),
            pltpu.async_copy(ips_i.at[l, pl.ds(b0, T)], pib[b], isem[b]),
        ]
        descs = []
        for r in range(IDX_ROWS):
            descs.append(pltpu.async_copy(
                asn_table.at[aidx_all.at[j, r]],
                gbuf[b].at[pl.ds(r * 128, 128)],
                gsem[b]))
        gather_descs[j] = descs

    def assemble(j):
        b = j % NBUF

        def group_body(g, carry):
            t0 = g * 16
            cols = iota + t0
            sct[b][0, pl.ds(t0, 16)] = latb[b][pl.ds(t0, 16)]
            sct[b][1, pl.ds(t0, 16)] = lonb[b][pl.ds(t0, 16)]
            for c in range(ASN_D):
                v = plsc.load_gather(gbuf[b],
                                     [cols, jnp.full((16,), 2 + c,
                                                     jnp.int32)])
                sct[b][2 + c, pl.ds(t0, 16)] = v
            gi = gib[b][pl.ds(t0, 16)]
            for c in range(GEO_D):
                v = plsc.load_gather(geo_tab_v,
                                     [gi, jnp.full((16,), c, jnp.int32)])
                sct[b][21 + c, pl.ds(t0, 16)] = v
            pi = pib[b][pl.ds(t0, 16)]
            for c in range(3):
                v = plsc.load_gather(ips_tab_v,
                                     [pi, jnp.full((16,), c, jnp.int32)])
                sct[b][29 + c, pl.ds(t0, 16)] = v
            return carry

        lax.fori_loop(0, G, group_body, 0)

    stage_and_fire(0)
    stage_and_fire(1)
    for j in range(PER_W):
        b = j % NBUF
        if j + 2 < PER_W:
            if j + 2 >= NBUF:
                out_descs[j - 1].wait()
            stage_and_fire(j + 2)
        for d in gather_descs[j]:
            d.wait()
        for d in in_descs[j]:
            d.wait()
        assemble(j)
        k = kc0 + j
        l = k // CPL
        b0 = pl.multiple_of((k % CPL) * T, 8)
        out_descs[j] = pltpu.async_copy(
            sct[b], out.at[l, :, pl.ds(b0, T)], osem[b])
    for j in range(PER_W - 2, PER_W):
        out_descs[j].wait()


@jax.jit
def _run(asn_table, geo_table, ips_table, lat, lon, geo_i, ips_i, asn_idx):
    mesh = plsc.VectorSubcoreMesh(core_axis_name="c", subcore_axis_name="s")
    return pl.kernel(
        _body_fixed,
        out_type=jax.ShapeDtypeStruct((L, OUT_D, B), jnp.float32),
        mesh=mesh,
        compiler_params=pltpu.CompilerParams(
            needs_layout_passes=False, use_tc_tiling_on_sc=False),
        scratch_types=(
            [pltpu.VMEM((T,), jnp.float32) for _ in range(NBUF)]
            + [pltpu.VMEM((T,), jnp.float32) for _ in range(NBUF)]
            + [pltpu.VMEM((T,), jnp.int32) for _ in range(NBUF)]
            + [pltpu.VMEM((T,), jnp.int32) for _ in range(NBUF)]
            + [pltpu.VMEM((PER_W, IDX_ROWS, 128), jnp.int32)]
            + [pltpu.VMEM((T, 24), jnp.float32) for _ in range(NBUF)]
            + [pltpu.VMEM((OUT_D, T), jnp.float32) for _ in range(NBUF)]
            + [pltpu.VMEM((GEO_V, GEO_D), jnp.float32),
               pltpu.VMEM((8, 4), jnp.float32)]
            + [pltpu.SemaphoreType.DMA for _ in range(3 * NBUF)]
        ),
    )(asn_table, geo_table, ips_table, lat, lon, geo_i, ips_i, asn_idx)


def kernel(x_lat, x_long, x_asn, x_geo_cc, x_ip_source,
           asn_table, geo_cc_table, ip_source_table):
    lat = x_lat.reshape(B, L).T
    lon = x_long.reshape(B, L).T
    geo_i = x_geo_cc.T
    ips_i = x_ip_source.T
    asn_idx = x_asn.astype(jnp.int32).T.reshape(NCHUNKS, IDX_ROWS, 128)
    ips_pad = jnp.pad(ip_source_table, ((0, 3), (0, 1)))
    asn_pad = jnp.pad(asn_table, ((0, 0), (2, 3)))
    asn_pad = jex_layout.with_layout_constraint(
        asn_pad, jex_layout.Layout((1, 0), tiling=()))
    out_t = _run(asn_pad, geo_cc_table, ips_pad, lat, lon, geo_i, ips_i,
                 asn_idx)
    return jnp.transpose(out_t, (2, 0, 1))
